# Initial kernel scaffold; baseline (speedup 1.0000x reference)
import jax
import jax.numpy as jnp
from jax.experimental import pallas as pl


def kernel(x, edge_index, W0, W1, W2):
    raise NotImplementedError("write your pallas kernel here")


# speedup vs baseline: 5.3244x; 5.3244x over previous
import functools

import jax
import jax.numpy as jnp
from jax import lax
from jax.experimental import pallas as pl
from jax.experimental.pallas import tpu as pltpu
from jax.experimental.pallas import tpu_sc as plsc

_N = 10000
_E = 160000
_D = 256
_DH = _D // 2
_NC = 2
_NS = 16
_CHUNK = 80
_EPT = _E // _NS
_ROWS = _EPT // _CHUNK
_EPW = _E // (_NC * _NS)
_EPW_P = 5008
_NPT8 = 624
_NPT_LAST = _N - (_NS - 1) * _NPT8
_RB = 1000

_mesh = plsc.VectorSubcoreMesh(core_axis_name="c", subcore_axis_name="s",
                               num_cores=_NC, num_subcores=_NS)


def _sc_deg_body(dst_hbm, out, dst_v, deg_v):
    c = lax.axis_index("c")
    s = lax.axis_index("s")
    wid = s * _NC + c

    zf = jnp.zeros((16,), jnp.float32)

    @pl.loop(0, _N // 16)
    def _(i):
        deg_v[pl.ds(i * 16, 16)] = zf

    dst_v[pl.ds(_EPW_P - 16, 16)] = jnp.zeros((16,), jnp.int32)
    pltpu.sync_copy(dst_hbm.at[pl.ds(wid * _EPW, _EPW)],
                    dst_v.at[pl.ds(0, _EPW)])

    ones = jnp.ones((16,), jnp.float32)

    @pl.loop(0, _EPW // 16)
    def _(i):
        idx = dst_v[pl.ds(i * 16, 16)]
        plsc.addupdate_scatter(deg_v, [idx], ones)

    tail = dst_v[pl.ds(_EPW // 16 * 16, 16)]
    mask = lax.iota(jnp.int32, 16) < (_EPW - _EPW // 16 * 16)
    plsc.addupdate_scatter(deg_v, [tail], ones, mask=mask)

    pltpu.sync_copy(deg_v, out.at[pl.ds(wid * _N, _N)])


_sc_deg = functools.partial(
    pl.kernel,
    out_type=jax.ShapeDtypeStruct((_NC * _NS * _N,), jnp.float32),
    mesh=_mesh,
    scratch_types=[
        pltpu.VMEM((_EPW_P,), jnp.int32),
        pltpu.VMEM((_N,), jnp.float32),
    ],
    compiler_params=pltpu.CompilerParams(needs_layout_passes=False),
)(_sc_deg_body)


def _sc_agg_body(h2, src_hbm, dst_hbm, zeros, out, src_v, dst_v, rowbuf,
                 agg_sh, sem):
    c = lax.axis_index("c")
    s = lax.axis_index("s")

    @pl.when(s < _NS - 1)
    def _():
        pltpu.sync_copy(zeros.at[pl.ds(s * _NPT8, _NPT8)],
                        agg_sh.at[pl.ds(s * _NPT8, _NPT8)])

    @pl.when(s == _NS - 1)
    def _():
        pltpu.sync_copy(zeros.at[pl.ds((_NS - 1) * _NPT8, _NPT_LAST)],
                        agg_sh.at[pl.ds((_NS - 1) * _NPT8, _NPT_LAST)])

    pltpu.sync_copy(src_hbm.at[pl.ds(s * _EPT, _EPT)], src_v)
    pltpu.sync_copy(dst_hbm.at[pl.ds(s * _EPT, _EPT)], dst_v)
    plsc.subcore_barrier()

    h_c = h2.at[c]

    @pl.loop(0, _ROWS)
    def _(j):
        sl = pl.ds(j * _CHUNK, _CHUNK)
        pltpu.async_copy(h_c.at[src_v.at[sl]], rowbuf, sem).wait()
        pltpu.sync_copy(rowbuf, agg_sh.at[dst_v.at[sl]], add=True)

    plsc.subcore_barrier()

    @pl.when(s < _NS - 1)
    def _():
        pltpu.sync_copy(agg_sh.at[pl.ds(s * _NPT8, _NPT8)],
                        out.at[c, pl.ds(s * _NPT8, _NPT8)])

    @pl.when(s == _NS - 1)
    def _():
        pltpu.sync_copy(agg_sh.at[pl.ds((_NS - 1) * _NPT8, _NPT_LAST)],
                        out.at[c, pl.ds((_NS - 1) * _NPT8, _NPT_LAST)])


_sc_agg = functools.partial(
    pl.kernel,
    out_type=jax.ShapeDtypeStruct((_NC, _N, _DH), jnp.float32),
    mesh=_mesh,
    scratch_types=[
        pltpu.VMEM((_EPT,), jnp.int32),
        pltpu.VMEM((_EPT,), jnp.int32),
        pltpu.VMEM((_CHUNK, _DH), jnp.float32),
        pltpu.VMEM_SHARED((_N, _DH), jnp.float32),
        pltpu.SemaphoreType.DMA,
    ],
)(_sc_agg_body)


def _mm0_body(x_ref, w_ref, h2_ref):
    h = jnp.dot(x_ref[...], w_ref[...], preferred_element_type=jnp.float32)
    h2_ref[0] = h[:, :_DH]
    h2_ref[1] = h[:, _DH:]


def _inv_deg(dpt_ref):
    deg = jnp.sum(dpt_ref[...], axis=1, keepdims=True)
    return 1.0 / jnp.maximum(deg, 1.0)


def _upd_body(x_ref, agg_ref, dpt_ref, w_ref, xn_ref, h2_ref):
    inv = _inv_deg(dpt_ref)
    agg = jnp.concatenate([agg_ref[0], agg_ref[1]], axis=-1)
    xn = x_ref[...] + jnp.maximum(agg * inv, 0.0)
    xn_ref[...] = xn
    h = jnp.dot(xn, w_ref[...], preferred_element_type=jnp.float32)
    h2_ref[0] = h[:, :_DH]
    h2_ref[1] = h[:, _DH:]


def _fin_body(x_ref, agg_ref, dpt_ref, out_ref):
    inv = _inv_deg(dpt_ref)
    agg = jnp.concatenate([agg_ref[0], agg_ref[1]], axis=-1)
    xn = x_ref[...] + jnp.maximum(agg * inv, 0.0)
    nrm = jnp.sqrt(jnp.sum(xn * xn, axis=1, keepdims=True))
    out_ref[...] = xn / jnp.maximum(nrm, 1e-12)


_x_spec = pl.BlockSpec((_RB, _D), lambda i: (i, 0))
_agg_spec = pl.BlockSpec((_NC, _RB, _DH), lambda i: (0, i, 0))
_dpt_spec = pl.BlockSpec((_RB, _NC * _NS), lambda i: (i, 0))
_w_spec = pl.BlockSpec((_D, _D), lambda i: (0, 0))
_h2_spec = pl.BlockSpec((_NC, _RB, _DH), lambda i: (0, i, 0))

_mm0 = pl.pallas_call(
    _mm0_body,
    grid=(_N // _RB,),
    in_specs=[_x_spec, _w_spec],
    out_specs=_h2_spec,
    out_shape=jax.ShapeDtypeStruct((_NC, _N, _DH), jnp.float32),
)

_upd = pl.pallas_call(
    _upd_body,
    grid=(_N // _RB,),
    in_specs=[_x_spec, _agg_spec, _dpt_spec, _w_spec],
    out_specs=[_x_spec, _h2_spec],
    out_shape=[jax.ShapeDtypeStruct((_N, _D), jnp.float32),
               jax.ShapeDtypeStruct((_NC, _N, _DH), jnp.float32)],
)

_fin = pl.pallas_call(
    _fin_body,
    grid=(_N // _RB,),
    in_specs=[_x_spec, _agg_spec, _dpt_spec],
    out_specs=_x_spec,
    out_shape=jax.ShapeDtypeStruct((_N, _D), jnp.float32),
)


def kernel(x, edge_index, W0, W1, W2):
    src = edge_index[0]
    dst = edge_index[1]
    zeros = jnp.zeros((_N, _DH), jnp.float32)

    deg_flat = _sc_deg(dst)
    dpt = deg_flat.reshape(_NC * _NS, _N).T

    h2 = _mm0(x, W0)
    agg = _sc_agg(h2, src, dst, zeros)
    x1, h2 = _upd(x, agg, dpt, W1)
    agg = _sc_agg(h2, src, dst, zeros)
    x2, h2 = _upd(x1, agg, dpt, W2)
    agg = _sc_agg(h2, src, dst, zeros)
    return _fin(x2, agg, dpt)


# speedup vs baseline: 6.7658x; 1.2707x over previous
import functools

import jax
import jax.numpy as jnp
from jax import lax
from jax.experimental import pallas as pl
from jax.experimental.pallas import tpu as pltpu
from jax.experimental.pallas import tpu_sc as plsc

_N = 10000
_E = 160000
_D = 256
_DH = _D // 2
_NC = 2
_NS = 16
_CHUNK = 80
_EPT = _E // _NS
_ROWS = _EPT // _CHUNK
_EPW = _E // (_NC * _NS)
_EPW_P = 5008
_NPT8 = 624
_NPT_LAST = _N - (_NS - 1) * _NPT8
_RB = 1000

_mesh = plsc.VectorSubcoreMesh(core_axis_name="c", subcore_axis_name="s",
                               num_cores=_NC, num_subcores=_NS)


def _sc_deg_body(dst_hbm, out, dst_v, deg_v):
    c = lax.axis_index("c")
    s = lax.axis_index("s")
    wid = s * _NC + c

    zf = jnp.zeros((16,), jnp.float32)

    @pl.loop(0, _N // 16)
    def _(i):
        deg_v[pl.ds(i * 16, 16)] = zf

    dst_v[pl.ds(_EPW_P - 16, 16)] = jnp.zeros((16,), jnp.int32)
    pltpu.sync_copy(dst_hbm.at[pl.ds(wid * _EPW, _EPW)],
                    dst_v.at[pl.ds(0, _EPW)])

    ones = jnp.ones((16,), jnp.float32)

    @pl.loop(0, _EPW // 16)
    def _(i):
        idx = dst_v[pl.ds(i * 16, 16)]
        plsc.addupdate_scatter(deg_v, [idx], ones)

    tail = dst_v[pl.ds(_EPW // 16 * 16, 16)]
    mask = lax.iota(jnp.int32, 16) < (_EPW - _EPW // 16 * 16)
    plsc.addupdate_scatter(deg_v, [tail], ones, mask=mask)

    pltpu.sync_copy(deg_v, out.at[pl.ds(wid * _N, _N)])


_sc_deg = functools.partial(
    pl.kernel,
    out_type=jax.ShapeDtypeStruct((_NC * _NS * _N,), jnp.float32),
    mesh=_mesh,
    scratch_types=[
        pltpu.VMEM((_EPW_P,), jnp.int32),
        pltpu.VMEM((_N,), jnp.float32),
    ],
    compiler_params=pltpu.CompilerParams(needs_layout_passes=False),
)(_sc_deg_body)


def _sc_agg_body(h2, src_hbm, dst_hbm, zeros, out, src_v, dst_v, rowbuf,
                 rowbuf2, agg_sh, sem, sem2):
    c = lax.axis_index("c")
    s = lax.axis_index("s")

    @pl.when(s < _NS - 1)
    def _():
        pltpu.sync_copy(zeros.at[pl.ds(s * _NPT8, _NPT8)],
                        agg_sh.at[pl.ds(s * _NPT8, _NPT8)])

    @pl.when(s == _NS - 1)
    def _():
        pltpu.sync_copy(zeros.at[pl.ds((_NS - 1) * _NPT8, _NPT_LAST)],
                        agg_sh.at[pl.ds((_NS - 1) * _NPT8, _NPT_LAST)])

    pltpu.sync_copy(src_hbm.at[pl.ds(s * _EPT, _EPT)], src_v)
    pltpu.sync_copy(dst_hbm.at[pl.ds(s * _EPT, _EPT)], dst_v)
    plsc.subcore_barrier()

    h_c = h2.at[c]
    bufs = (rowbuf, rowbuf2)
    sems = (sem, sem2)

    def _start_gather(j, buf, sm):
        @pl.when(j < _ROWS)
        def _():
            pltpu.async_copy(h_c.at[src_v.at[pl.ds(j * _CHUNK, _CHUNK)]],
                             buf, sm)

    def _wait_gather(j, buf, sm):
        pltpu.make_async_copy(
            h_c.at[src_v.at[pl.ds(j * _CHUNK, _CHUNK)]], buf, sm).wait()

    def _scatter(j, buf):
        pltpu.sync_copy(buf, agg_sh.at[dst_v.at[pl.ds(j * _CHUNK, _CHUNK)]],
                        add=True)

    _start_gather(0, bufs[0], sems[0])

    @pl.loop(0, _ROWS, step=2)
    def _(j):
        _wait_gather(j, bufs[0], sems[0])
        _start_gather(j + 1, bufs[1], sems[1])
        _scatter(j, bufs[0])

        @pl.when(j + 1 < _ROWS)
        def _():
            _wait_gather(j + 1, bufs[1], sems[1])
            _start_gather(j + 2, bufs[0], sems[0])
            _scatter(j + 1, bufs[1])

    plsc.subcore_barrier()

    @pl.when(s < _NS - 1)
    def _():
        pltpu.sync_copy(agg_sh.at[pl.ds(s * _NPT8, _NPT8)],
                        out.at[c, pl.ds(s * _NPT8, _NPT8)])

    @pl.when(s == _NS - 1)
    def _():
        pltpu.sync_copy(agg_sh.at[pl.ds((_NS - 1) * _NPT8, _NPT_LAST)],
                        out.at[c, pl.ds((_NS - 1) * _NPT8, _NPT_LAST)])


_sc_agg = functools.partial(
    pl.kernel,
    out_type=jax.ShapeDtypeStruct((_NC, _N, _DH), jnp.float32),
    mesh=_mesh,
    scratch_types=[
        pltpu.VMEM((_EPT,), jnp.int32),
        pltpu.VMEM((_EPT,), jnp.int32),
        pltpu.VMEM((_CHUNK, _DH), jnp.float32),
        pltpu.VMEM((_CHUNK, _DH), jnp.float32),
        pltpu.VMEM_SHARED((_N, _DH), jnp.float32),
        pltpu.SemaphoreType.DMA,
        pltpu.SemaphoreType.DMA,
    ],
)(_sc_agg_body)


def _mm0_body(x_ref, w_ref, h2_ref):
    h = jnp.dot(x_ref[...], w_ref[...], preferred_element_type=jnp.float32)
    h2_ref[0] = h[:, :_DH]
    h2_ref[1] = h[:, _DH:]


def _inv_deg(dpt_ref):
    deg = jnp.sum(dpt_ref[...], axis=1, keepdims=True)
    return 1.0 / jnp.maximum(deg, 1.0)


def _upd_body(x_ref, agg_ref, dpt_ref, w_ref, xn_ref, h2_ref):
    inv = _inv_deg(dpt_ref)
    agg = jnp.concatenate([agg_ref[0], agg_ref[1]], axis=-1)
    xn = x_ref[...] + jnp.maximum(agg * inv, 0.0)
    xn_ref[...] = xn
    h = jnp.dot(xn, w_ref[...], preferred_element_type=jnp.float32)
    h2_ref[0] = h[:, :_DH]
    h2_ref[1] = h[:, _DH:]


def _fin_body(x_ref, agg_ref, dpt_ref, out_ref):
    inv = _inv_deg(dpt_ref)
    agg = jnp.concatenate([agg_ref[0], agg_ref[1]], axis=-1)
    xn = x_ref[...] + jnp.maximum(agg * inv, 0.0)
    nrm = jnp.sqrt(jnp.sum(xn * xn, axis=1, keepdims=True))
    out_ref[...] = xn / jnp.maximum(nrm, 1e-12)


_x_spec = pl.BlockSpec((_RB, _D), lambda i: (i, 0))
_agg_spec = pl.BlockSpec((_NC, _RB, _DH), lambda i: (0, i, 0))
_dpt_spec = pl.BlockSpec((_RB, _NC * _NS), lambda i: (i, 0))
_w_spec = pl.BlockSpec((_D, _D), lambda i: (0, 0))
_h2_spec = pl.BlockSpec((_NC, _RB, _DH), lambda i: (0, i, 0))

_mm0 = pl.pallas_call(
    _mm0_body,
    grid=(_N // _RB,),
    in_specs=[_x_spec, _w_spec],
    out_specs=_h2_spec,
    out_shape=jax.ShapeDtypeStruct((_NC, _N, _DH), jnp.float32),
)

_upd = pl.pallas_call(
    _upd_body,
    grid=(_N // _RB,),
    in_specs=[_x_spec, _agg_spec, _dpt_spec, _w_spec],
    out_specs=[_x_spec, _h2_spec],
    out_shape=[jax.ShapeDtypeStruct((_N, _D), jnp.float32),
               jax.ShapeDtypeStruct((_NC, _N, _DH), jnp.float32)],
)

_fin = pl.pallas_call(
    _fin_body,
    grid=(_N // _RB,),
    in_specs=[_x_spec, _agg_spec, _dpt_spec],
    out_specs=_x_spec,
    out_shape=jax.ShapeDtypeStruct((_N, _D), jnp.float32),
)


def kernel(x, edge_index, W0, W1, W2):
    src = edge_index[0]
    dst = edge_index[1]
    zeros = jnp.zeros((_N, _DH), jnp.float32)

    deg_flat = _sc_deg(dst)
    dpt = deg_flat.reshape(_NC * _NS, _N).T

    h2 = _mm0(x, W0)
    agg = _sc_agg(h2, src, dst, zeros)
    x1, h2 = _upd(x, agg, dpt, W1)
    agg = _sc_agg(h2, src, dst, zeros)
    x2, h2 = _upd(x1, agg, dpt, W2)
    agg = _sc_agg(h2, src, dst, zeros)
    return _fin(x2, agg, dpt)
